# Initial kernel scaffold; baseline (speedup 1.0000x reference)
import jax
import jax.numpy as jnp
from jax.experimental import pallas as pl


def kernel(x, edge_index, edge_attr, batch, nn1_W, nn1_b, root1, bias1, nn2_W, nn2_b, root2, bias2, lin1_W, lin1_b, lin2_W, lin2_b):
    raise NotImplementedError("write your pallas kernel here")


# speedup vs baseline: 1.9308x; 1.9308x over previous
import functools

import jax
import jax.numpy as jnp
from jax import lax
from jax.experimental import pallas as pl
from jax.experimental.pallas import tpu as pltpu
from jax.experimental.pallas import tpu_sc as plsc

N = 10000
E = 160000
D_NODE = 64
D_EDGE = 16
H = 32
G = 64

NC = 2
NS = 16
NW = NC * NS
ROWS_DMA = 128
E_PAD = 163840
PER_W = E_PAD // NW
K_DMA = PER_W // ROWS_DMA
GROUP = 1024
N_GROUPS = PER_W // GROUP
DMA_PER_GROUP = GROUP // ROWS_DMA
NP = 10240
ROWS_PER_TILE = NP // NS

@functools.lru_cache(maxsize=None)
def _sc_mesh():
    return plsc.VectorSubcoreMesh(
        core_axis_name="c", subcore_axis_name="s",
        num_cores=NC, num_subcores=NS)


@functools.lru_cache(maxsize=None)
def _make_gather(d_in):
    def body(table_hbm, src_hbm, out_hbm, idx_v, rows_v, sem):
        wid = lax.axis_index("s") * NC + lax.axis_index("c")
        base = wid * PER_W
        pltpu.sync_copy(src_hbm.at[wid], idx_v)

        def group(g, carry):
            descs = []
            for j in range(DMA_PER_GROUP):
                descs.append(pltpu.async_copy(
                    table_hbm.at[idx_v.at[g * DMA_PER_GROUP + j]],
                    rows_v.at[pl.ds(j * ROWS_DMA, ROWS_DMA)], sem))
            for dsc in descs:
                dsc.wait()
            pltpu.sync_copy(rows_v, out_hbm.at[pl.ds(base + g * GROUP, GROUP)])
            return carry

        lax.fori_loop(0, N_GROUPS, group, 0)

    return pl.kernel(
        body,
        out_type=jax.ShapeDtypeStruct((E_PAD, d_in), jnp.float32),
        mesh=_sc_mesh(),
        compiler_params=pltpu.CompilerParams(use_tc_tiling_on_sc=False),
        scratch_types=[
            pltpu.VMEM((K_DMA, ROWS_DMA), jnp.int32),
            pltpu.VMEM((GROUP, d_in), jnp.float32),
            pltpu.SemaphoreType.DMA,
        ],
    )


def _scatter_body(msg_hbm, dst_hbm, zeros_hbm, out_hbm, idx_v, msg_v, acc_sh, sem):
    cid = lax.axis_index("c")
    sid = lax.axis_index("s")
    wid = sid * NC + cid
    base = wid * PER_W
    row0 = sid * ROWS_PER_TILE
    pltpu.sync_copy(zeros_hbm.at[pl.ds(row0, ROWS_PER_TILE)],
                    acc_sh.at[pl.ds(row0, ROWS_PER_TILE)])
    pltpu.sync_copy(dst_hbm.at[wid], idx_v)
    plsc.subcore_barrier()

    def group(g, carry):
        pltpu.sync_copy(msg_hbm.at[pl.ds(base + g * GROUP, GROUP)], msg_v)
        for j in range(DMA_PER_GROUP):
            pltpu.sync_copy(msg_v.at[pl.ds(j * ROWS_DMA, ROWS_DMA)],
                            acc_sh.at[idx_v.at[g * DMA_PER_GROUP + j]],
                            add=True)
        return carry

    lax.fori_loop(0, N_GROUPS, group, 0)
    plsc.subcore_barrier()
    pltpu.sync_copy(acc_sh.at[pl.ds(row0, ROWS_PER_TILE)],
                    out_hbm.at[pl.ds(cid * NP + row0, ROWS_PER_TILE)])


@functools.lru_cache(maxsize=None)
def _make_scatter():
    return pl.kernel(
        _scatter_body,
        out_type=jax.ShapeDtypeStruct((NC * NP, H), jnp.float32),
        mesh=_sc_mesh(),
        compiler_params=pltpu.CompilerParams(use_tc_tiling_on_sc=False),
        scratch_types=[
            pltpu.VMEM((K_DMA, ROWS_DMA), jnp.int32),
            pltpu.VMEM((GROUP, H), jnp.float32),
            pltpu.VMEM_SHARED((NP, H), jnp.float32),
            pltpu.SemaphoreType.DMA,
        ],
    )


BE = 1024


def _msg_body(d_in, xj_ref, ea_ref, w_ref, out_ref):
    xj = xj_ref[...]
    ea = ea_ref[...]
    z = jnp.concatenate(
        [ea[:, k:k + 1] * xj for k in range(D_EDGE + 1)], axis=1)
    out_ref[...] = jnp.dot(z, w_ref[...], preferred_element_type=jnp.float32)


def _make_msg(d_in):
    kd = (D_EDGE + 1) * d_in
    return pl.pallas_call(
        functools.partial(_msg_body, d_in),
        grid=(E_PAD // BE,),
        in_specs=[
            pl.BlockSpec((BE, d_in), lambda i: (i, 0)),
            pl.BlockSpec((BE, D_EDGE + 1), lambda i: (i, 0)),
            pl.BlockSpec((kd, H), lambda i: (0, 0)),
        ],
        out_specs=pl.BlockSpec((BE, H), lambda i: (i, 0)),
        out_shape=jax.ShapeDtypeStruct((E_PAD, H), jnp.float32),
    )


_msg64 = _make_msg(D_NODE)
_msg32 = _make_msg(H)


def _update_body(x_ref, a0_ref, a1_ref, root_ref, bias_ref, out_ref):
    h = (jnp.dot(x_ref[...], root_ref[...], preferred_element_type=jnp.float32)
         + a0_ref[...] + a1_ref[...] + bias_ref[...])
    out_ref[...] = jnp.maximum(h, 0.0)


def _make_update(d_in):
    return pl.pallas_call(
        _update_body,
        out_shape=jax.ShapeDtypeStruct((N, H), jnp.float32),
    )


_update1 = _make_update(D_NODE)


def _final_body(h1_ref, a0_ref, a1_ref, root_ref, bias_ref, batch_ref,
                l1w_ref, l1b_ref, l2w_ref, l2b_ref, out_ref):
    h2 = (jnp.dot(h1_ref[...], root_ref[...],
                  preferred_element_type=jnp.float32)
          + a0_ref[...] + a1_ref[...] + bias_ref[...])
    gids = jnp.broadcast_to(batch_ref[...], (G, N))
    oht = (lax.broadcasted_iota(jnp.int32, (G, N), 0) == gids
           ).astype(jnp.float32)
    sums = jnp.dot(oht, h2, preferred_element_type=jnp.float32)
    cnt = jnp.sum(oht, axis=1, keepdims=True)
    pooled = sums / jnp.maximum(cnt, 1.0)
    z = jnp.maximum(
        jnp.dot(pooled, l1w_ref[...], preferred_element_type=jnp.float32)
        + l1b_ref[...], 0.0)
    out_ref[...] = jax.nn.sigmoid(
        jnp.dot(z, l2w_ref[...], preferred_element_type=jnp.float32)
        + l2b_ref[...])


_final = pl.pallas_call(
    _final_body,
    out_shape=jax.ShapeDtypeStruct((G, 1), jnp.float32),
)


def kernel(x, edge_index, edge_attr, batch,
           nn1_W, nn1_b, root1, bias1,
           nn2_W, nn2_b, root2, bias2,
           lin1_W, lin1_b, lin2_W, lin2_b):
    src = edge_index[0].astype(jnp.int32)
    dst = edge_index[1].astype(jnp.int32)
    pad = E_PAD - E
    src_p = jnp.concatenate([src, jnp.zeros((pad,), jnp.int32)]
                            ).reshape(NW, K_DMA, ROWS_DMA)
    dst_p = jnp.concatenate([dst, jnp.full((pad,), NP - 1, jnp.int32)]
                            ).reshape(NW, K_DMA, ROWS_DMA)
    ea1 = jnp.concatenate([edge_attr, jnp.ones((E, 1), jnp.float32)], axis=1)
    ea1 = jnp.concatenate(
        [ea1, jnp.zeros((pad, D_EDGE + 1), jnp.float32)], axis=0)
    wall1 = jnp.concatenate([nn1_W.reshape(D_EDGE * D_NODE, H),
                             nn1_b.reshape(D_NODE, H)], axis=0)
    wall2 = jnp.concatenate([nn2_W.reshape(D_EDGE * H, H),
                             nn2_b.reshape(H, H)], axis=0)
    zeros_np = jnp.zeros((NP, H), jnp.float32)
    batch2d = batch.astype(jnp.int32).reshape(1, N)

    xj = _make_gather(D_NODE)(x, src_p)
    msg1 = _msg64(xj, ea1, wall1)
    agg1 = _make_scatter()(msg1, dst_p, zeros_np)
    h1 = _update1(x, agg1[:N], agg1[NP:NP + N], root1, bias1.reshape(1, H))

    hj = _make_gather(H)(h1, src_p)
    msg2 = _msg32(hj, ea1, wall2)
    agg2 = _make_scatter()(msg2, dst_p, zeros_np)

    return _final(h1, agg2[:N], agg2[NP:NP + N], root2, bias2.reshape(1, H),
                  batch2d, lin1_W, lin1_b.reshape(1, H // 2),
                  lin2_W, lin2_b.reshape(1, 1))


# speedup vs baseline: 3.0632x; 1.5865x over previous
import functools

import jax
import jax.numpy as jnp
from jax import lax
from jax.experimental import pallas as pl
from jax.experimental.pallas import tpu as pltpu
from jax.experimental.pallas import tpu_sc as plsc

N = 10000
E = 160000
D_NODE = 64
D_EDGE = 16
H = 32
G = 64

NC = 2
NS = 16
NW = NC * NS
ROWS_DMA = 128
E_PAD = 163840
PER_W = E_PAD // NW
K_DMA = PER_W // ROWS_DMA
GROUP = 1024
N_GROUPS = PER_W // GROUP
DMA_PER_GROUP = GROUP // ROWS_DMA
NP = 10240
ROWS_PER_TILE = NP // NS

@functools.lru_cache(maxsize=None)
def _sc_mesh():
    return plsc.VectorSubcoreMesh(
        core_axis_name="c", subcore_axis_name="s",
        num_cores=NC, num_subcores=NS)


@functools.lru_cache(maxsize=None)
def _make_gather(d_in):
    def body(table_hbm, src_hbm, out_hbm, idx_v, rows_v, sem):
        wid = lax.axis_index("s") * NC + lax.axis_index("c")
        base = wid * PER_W
        pltpu.sync_copy(src_hbm.at[wid], idx_v)

        def group(g, carry):
            descs = []
            for j in range(DMA_PER_GROUP):
                descs.append(pltpu.async_copy(
                    table_hbm.at[idx_v.at[g * DMA_PER_GROUP + j]],
                    rows_v.at[pl.ds(j * ROWS_DMA, ROWS_DMA)], sem))
            for dsc in descs:
                dsc.wait()
            pltpu.sync_copy(rows_v, out_hbm.at[pl.ds(base + g * GROUP, GROUP)])
            return carry

        lax.fori_loop(0, N_GROUPS, group, 0)

    return pl.kernel(
        body,
        out_type=jax.ShapeDtypeStruct((E_PAD, d_in), jnp.float32),
        mesh=_sc_mesh(),
        compiler_params=pltpu.CompilerParams(use_tc_tiling_on_sc=False),
        scratch_types=[
            pltpu.VMEM((K_DMA, ROWS_DMA), jnp.int32),
            pltpu.VMEM((GROUP, d_in), jnp.float32),
            pltpu.SemaphoreType.DMA,
        ],
    )


def _scatter_body(msg_hbm, dst_hbm, zeros_hbm, out_hbm, idx_v, msg_v, acc_sh, sem):
    cid = lax.axis_index("c")
    sid = lax.axis_index("s")
    wid = sid * NC + cid
    base = wid * PER_W
    row0 = sid * ROWS_PER_TILE
    pltpu.sync_copy(zeros_hbm.at[pl.ds(row0, ROWS_PER_TILE)],
                    acc_sh.at[pl.ds(row0, ROWS_PER_TILE)])
    pltpu.sync_copy(dst_hbm.at[wid], idx_v)
    plsc.subcore_barrier()

    def group(g, carry):
        pltpu.sync_copy(msg_hbm.at[pl.ds(base + g * GROUP, GROUP)], msg_v)
        for j in range(DMA_PER_GROUP):
            pltpu.sync_copy(msg_v.at[pl.ds(j * ROWS_DMA, ROWS_DMA)],
                            acc_sh.at[idx_v.at[g * DMA_PER_GROUP + j]],
                            add=True)
        return carry

    lax.fori_loop(0, N_GROUPS, group, 0)
    plsc.subcore_barrier()
    pltpu.sync_copy(acc_sh.at[pl.ds(row0, ROWS_PER_TILE)],
                    out_hbm.at[pl.ds(cid * NP + row0, ROWS_PER_TILE)])


@functools.lru_cache(maxsize=None)
def _make_scatter():
    return pl.kernel(
        _scatter_body,
        out_type=jax.ShapeDtypeStruct((NC * NP, H), jnp.float32),
        mesh=_sc_mesh(),
        compiler_params=pltpu.CompilerParams(use_tc_tiling_on_sc=False),
        scratch_types=[
            pltpu.VMEM((K_DMA, ROWS_DMA), jnp.int32),
            pltpu.VMEM((GROUP, H), jnp.float32),
            pltpu.VMEM_SHARED((NP, H), jnp.float32),
            pltpu.SemaphoreType.DMA,
        ],
    )


BE = 1024


NK = 32
KW = NK * H


def _msg_body(xj_ref, ea_ref, w_ref, r_ref, out_ref):
    y = jnp.dot(xj_ref[...].astype(jnp.bfloat16), w_ref[...],
                preferred_element_type=jnp.float32)
    c = jnp.dot(ea_ref[...].astype(jnp.bfloat16), r_ref[...],
                preferred_element_type=jnp.float32)
    t = c * y
    t = t[:, :512] + t[:, 512:]
    t = t[:, :256] + t[:, 256:]
    t = t[:, :128] + t[:, 128:]
    t = t[:, :64] + t[:, 64:]
    out_ref[...] = t[:, :H] + t[:, H:]


def _make_msg(d_in):
    return pl.pallas_call(
        _msg_body,
        grid=(E_PAD // BE,),
        in_specs=[
            pl.BlockSpec((BE, d_in), lambda i: (i, 0)),
            pl.BlockSpec((BE, NK), lambda i: (i, 0)),
            pl.BlockSpec((d_in, KW), lambda i: (0, 0)),
            pl.BlockSpec((NK, KW), lambda i: (0, 0)),
        ],
        out_specs=pl.BlockSpec((BE, H), lambda i: (i, 0)),
        out_shape=jax.ShapeDtypeStruct((E_PAD, H), jnp.float32),
    )


_msg64 = _make_msg(D_NODE)
_msg32 = _make_msg(H)


def _update_body(x_ref, a0_ref, a1_ref, root_ref, bias_ref, out_ref):
    h = (jnp.dot(x_ref[...], root_ref[...], preferred_element_type=jnp.float32)
         + a0_ref[...] + a1_ref[...] + bias_ref[...])
    out_ref[...] = jnp.maximum(h, 0.0)


def _make_update(d_in):
    return pl.pallas_call(
        _update_body,
        out_shape=jax.ShapeDtypeStruct((N, H), jnp.float32),
    )


_update1 = _make_update(D_NODE)


def _final_body(h1_ref, a0_ref, a1_ref, root_ref, bias_ref, batch_ref,
                l1w_ref, l1b_ref, l2w_ref, l2b_ref, out_ref):
    h2 = (jnp.dot(h1_ref[...], root_ref[...],
                  preferred_element_type=jnp.float32)
          + a0_ref[...] + a1_ref[...] + bias_ref[...])
    gids = jnp.broadcast_to(batch_ref[...], (G, N))
    oht = (lax.broadcasted_iota(jnp.int32, (G, N), 0) == gids
           ).astype(jnp.float32)
    sums = jnp.dot(oht, h2, preferred_element_type=jnp.float32)
    cnt = jnp.sum(oht, axis=1, keepdims=True)
    pooled = sums / jnp.maximum(cnt, 1.0)
    z = jnp.maximum(
        jnp.dot(pooled, l1w_ref[...], preferred_element_type=jnp.float32)
        + l1b_ref[...], 0.0)
    out_ref[...] = jax.nn.sigmoid(
        jnp.dot(z, l2w_ref[...], preferred_element_type=jnp.float32)
        + l2b_ref[...])


_final = pl.pallas_call(
    _final_body,
    out_shape=jax.ShapeDtypeStruct((G, 1), jnp.float32),
)


def kernel(x, edge_index, edge_attr, batch,
           nn1_W, nn1_b, root1, bias1,
           nn2_W, nn2_b, root2, bias2,
           lin1_W, lin1_b, lin2_W, lin2_b):
    src = edge_index[0].astype(jnp.int32)
    dst = edge_index[1].astype(jnp.int32)
    pad = E_PAD - E
    src_p = jnp.concatenate([src, jnp.zeros((pad,), jnp.int32)]
                            ).reshape(NW, K_DMA, ROWS_DMA)
    dst_p = jnp.concatenate([dst, jnp.full((pad,), NP - 1, jnp.int32)]
                            ).reshape(NW, K_DMA, ROWS_DMA)
    ea1 = jnp.concatenate(
        [edge_attr, jnp.ones((E, 1), jnp.float32),
         jnp.zeros((E, NK - D_EDGE - 1), jnp.float32)], axis=1)
    ea1 = jnp.concatenate([ea1, jnp.zeros((pad, NK), jnp.float32)], axis=0)

    def wcat(nnW, nnb, d_in):
        w = nnW.reshape(D_EDGE, d_in, H).transpose(1, 0, 2).reshape(d_in, -1)
        w = jnp.concatenate([w, nnb.reshape(d_in, H)], axis=1)
        return jnp.pad(w, ((0, 0), (0, KW - w.shape[1]))).astype(jnp.bfloat16)

    wall1 = wcat(nn1_W, nn1_b, D_NODE)
    wall2 = wcat(nn2_W, nn2_b, H)
    rmat = jnp.repeat(jnp.eye(NK, dtype=jnp.bfloat16), H, axis=1)
    zeros_np = jnp.zeros((NP, H), jnp.float32)
    batch2d = batch.astype(jnp.int32).reshape(1, N)

    xj = _make_gather(D_NODE)(x, src_p)
    msg1 = _msg64(xj, ea1, wall1, rmat)
    agg1 = _make_scatter()(msg1, dst_p, zeros_np)
    h1 = _update1(x, agg1[:N], agg1[NP:NP + N], root1, bias1.reshape(1, H))

    hj = _make_gather(H)(h1, src_p)
    msg2 = _msg32(hj, ea1, wall2, rmat)
    agg2 = _make_scatter()(msg2, dst_p, zeros_np)

    return _final(h1, agg2[:N], agg2[NP:NP + N], root2, bias2.reshape(1, H),
                  batch2d, lin1_W, lin1_b.reshape(1, H // 2),
                  lin2_W, lin2_b.reshape(1, 1))


# speedup vs baseline: 3.9643x; 1.2942x over previous
import functools

import jax
import jax.numpy as jnp
from jax import lax
from jax.experimental import pallas as pl
from jax.experimental.pallas import tpu as pltpu
from jax.experimental.pallas import tpu_sc as plsc

N = 10000
E = 160000
D_NODE = 64
D_EDGE = 16
H = 32
G = 64

NC = 2
NS = 16
NW = NC * NS
ROWS_DMA = 128
E_PAD = 163840
PER_W = E_PAD // NW
K_DMA = PER_W // ROWS_DMA
GROUP = 1024
N_GROUPS = PER_W // GROUP
DMA_PER_GROUP = GROUP // ROWS_DMA
NP = 10240
ROWS_PER_TILE = NP // NS

@functools.lru_cache(maxsize=None)
def _sc_mesh():
    return plsc.VectorSubcoreMesh(
        core_axis_name="c", subcore_axis_name="s",
        num_cores=NC, num_subcores=NS)


@functools.lru_cache(maxsize=None)
def _make_gather(d_in):
    group_rows = 32768 // d_in
    dpg = group_rows // ROWS_DMA
    ngr = PER_W // group_rows

    def body(table_hbm, src_hbm, out_hbm, idx_v, rows_v, semg, semo):
        wid = lax.axis_index("s") * NC + lax.axis_index("c")
        base = wid * PER_W
        pltpu.sync_copy(src_hbm.at[wid], idx_v)

        def fire(g, b):
            for j in range(dpg):
                pltpu.async_copy(
                    table_hbm.at[idx_v.at[g * dpg + j]],
                    rows_v.at[b, pl.ds(j * ROWS_DMA, ROWS_DMA)], semg)

        fire(0, 0)

        def group(g, carry):
            b = lax.rem(g, 2)
            nb = lax.rem(g + 1, 2)
            for j in range(dpg):
                pltpu.make_async_copy(
                    table_hbm.at[idx_v.at[j]],
                    rows_v.at[b, pl.ds(j * ROWS_DMA, ROWS_DMA)], semg).wait()

            @pl.when(g >= 1)
            def _():
                pltpu.make_async_copy(
                    rows_v.at[nb], out_hbm.at[pl.ds(base, group_rows)],
                    semo).wait()

            @pl.when(g + 1 < ngr)
            def _():
                fire(g + 1, nb)

            pltpu.async_copy(
                rows_v.at[b],
                out_hbm.at[pl.ds(base + g * group_rows, group_rows)], semo)
            return carry

        lax.fori_loop(0, ngr, group, 0)
        pltpu.make_async_copy(
            rows_v.at[lax.rem(ngr - 1, 2)],
            out_hbm.at[pl.ds(base, group_rows)], semo).wait()

    return pl.kernel(
        body,
        out_type=jax.ShapeDtypeStruct((E_PAD, d_in), jnp.float32),
        mesh=_sc_mesh(),
        compiler_params=pltpu.CompilerParams(use_tc_tiling_on_sc=False),
        scratch_types=[
            pltpu.VMEM((K_DMA, ROWS_DMA), jnp.int32),
            pltpu.VMEM((2, group_rows, d_in), jnp.float32),
            pltpu.SemaphoreType.DMA,
            pltpu.SemaphoreType.DMA,
        ],
    )


def _scatter_body(msg_hbm, dst_hbm, zeros_hbm, out_hbm,
                  idx_v, msg_v, acc_sh, seml, sems):
    cid = lax.axis_index("c")
    sid = lax.axis_index("s")
    wid = sid * NC + cid
    base = wid * PER_W
    row0 = sid * ROWS_PER_TILE
    pltpu.sync_copy(dst_hbm.at[wid], idx_v)
    pltpu.async_copy(msg_hbm.at[pl.ds(base, GROUP)], msg_v.at[0], seml)
    pltpu.sync_copy(zeros_hbm, acc_sh.at[pl.ds(row0, ROWS_PER_TILE)])
    plsc.subcore_barrier()

    def fire_scatters(g, b):
        for j in range(DMA_PER_GROUP):
            pltpu.async_copy(msg_v.at[b, pl.ds(j * ROWS_DMA, ROWS_DMA)],
                             acc_sh.at[idx_v.at[g * DMA_PER_GROUP + j]],
                             sems, add=True)

    def drain_scatters(b):
        for j in range(DMA_PER_GROUP):
            pltpu.make_async_copy(
                msg_v.at[b, pl.ds(j * ROWS_DMA, ROWS_DMA)],
                acc_sh.at[idx_v.at[j]], sems).wait()

    def group(g, carry):
        b = lax.rem(g, 2)
        nb = lax.rem(g + 1, 2)
        pltpu.make_async_copy(
            msg_hbm.at[pl.ds(base, GROUP)], msg_v.at[b], seml).wait()

        @pl.when(g >= 1)
        def _():
            drain_scatters(nb)

        @pl.when(g + 1 < N_GROUPS)
        def _():
            pltpu.async_copy(
                msg_hbm.at[pl.ds(base + (g + 1) * GROUP, GROUP)],
                msg_v.at[nb], seml)

        fire_scatters(g, b)
        return carry

    lax.fori_loop(0, N_GROUPS, group, 0)
    drain_scatters(lax.rem(N_GROUPS - 1, 2))
    plsc.subcore_barrier()
    pltpu.sync_copy(acc_sh.at[pl.ds(row0, ROWS_PER_TILE)],
                    out_hbm.at[pl.ds(cid * NP + row0, ROWS_PER_TILE)])


@functools.lru_cache(maxsize=None)
def _make_scatter():
    return pl.kernel(
        _scatter_body,
        out_type=jax.ShapeDtypeStruct((NC * NP, H), jnp.float32),
        mesh=_sc_mesh(),
        compiler_params=pltpu.CompilerParams(use_tc_tiling_on_sc=False),
        scratch_types=[
            pltpu.VMEM((K_DMA, ROWS_DMA), jnp.int32),
            pltpu.VMEM((2, GROUP, H), jnp.float32),
            pltpu.VMEM_SHARED((NP, H), jnp.float32),
            pltpu.SemaphoreType.DMA,
            pltpu.SemaphoreType.DMA,
        ],
    )


BE = 1024


NK = 32
KW = NK * H


def _msg_body(xj_ref, ea_ref, w_ref, r_ref, out_ref):
    y = jnp.dot(xj_ref[...].astype(jnp.bfloat16), w_ref[...],
                preferred_element_type=jnp.float32)
    c = jnp.dot(ea_ref[...].astype(jnp.bfloat16), r_ref[...],
                preferred_element_type=jnp.float32)
    t = c * y
    t = t[:, :512] + t[:, 512:]
    t = t[:, :256] + t[:, 256:]
    t = t[:, :128] + t[:, 128:]
    t = t[:, :64] + t[:, 64:]
    out_ref[...] = t[:, :H] + t[:, H:]


def _make_msg(d_in):
    return pl.pallas_call(
        _msg_body,
        grid=(E_PAD // BE,),
        in_specs=[
            pl.BlockSpec((BE, d_in), lambda i: (i, 0)),
            pl.BlockSpec((BE, NK), lambda i: (i, 0)),
            pl.BlockSpec((d_in, KW), lambda i: (0, 0)),
            pl.BlockSpec((NK, KW), lambda i: (0, 0)),
        ],
        out_specs=pl.BlockSpec((BE, H), lambda i: (i, 0)),
        out_shape=jax.ShapeDtypeStruct((E_PAD, H), jnp.float32),
    )


_msg64 = _make_msg(D_NODE)
_msg32 = _make_msg(H)


def _update_body(x_ref, agg_ref, root_ref, bias_ref, out_ref):
    agg = agg_ref[pl.ds(0, N)] + agg_ref[pl.ds(NP, N)]
    h = (jnp.dot(x_ref[...], root_ref[...], preferred_element_type=jnp.float32)
         + agg + bias_ref[...])
    out_ref[...] = jnp.maximum(h, 0.0)


def _make_update(d_in):
    return pl.pallas_call(
        _update_body,
        out_shape=jax.ShapeDtypeStruct((N, H), jnp.float32),
    )


_update1 = _make_update(D_NODE)


def _final_body(h1_ref, agg_ref, root_ref, bias_ref, batch_ref,
                l1w_ref, l1b_ref, l2w_ref, l2b_ref, out_ref):
    agg = agg_ref[pl.ds(0, N)] + agg_ref[pl.ds(NP, N)]
    h2 = (jnp.dot(h1_ref[...], root_ref[...],
                  preferred_element_type=jnp.float32)
          + agg + bias_ref[...])
    gids = jnp.broadcast_to(batch_ref[...], (G, N))
    oht = (lax.broadcasted_iota(jnp.int32, (G, N), 0) == gids
           ).astype(jnp.float32)
    sums = jnp.dot(oht, h2, preferred_element_type=jnp.float32)
    cnt = jnp.sum(oht, axis=1, keepdims=True)
    pooled = sums / jnp.maximum(cnt, 1.0)
    z = jnp.maximum(
        jnp.dot(pooled, l1w_ref[...], preferred_element_type=jnp.float32)
        + l1b_ref[...], 0.0)
    out_ref[...] = jax.nn.sigmoid(
        jnp.dot(z, l2w_ref[...], preferred_element_type=jnp.float32)
        + l2b_ref[...])


_final = pl.pallas_call(
    _final_body,
    out_shape=jax.ShapeDtypeStruct((G, 1), jnp.float32),
)


def kernel(x, edge_index, edge_attr, batch,
           nn1_W, nn1_b, root1, bias1,
           nn2_W, nn2_b, root2, bias2,
           lin1_W, lin1_b, lin2_W, lin2_b):
    src = edge_index[0].astype(jnp.int32)
    dst = edge_index[1].astype(jnp.int32)
    pad = E_PAD - E
    src_p = jnp.concatenate([src, jnp.zeros((pad,), jnp.int32)]
                            ).reshape(NW, K_DMA, ROWS_DMA)
    dst_p = jnp.concatenate([dst, jnp.full((pad,), NP - 1, jnp.int32)]
                            ).reshape(NW, K_DMA, ROWS_DMA)
    ea1 = jnp.concatenate(
        [edge_attr, jnp.ones((E, 1), jnp.float32),
         jnp.zeros((E, NK - D_EDGE - 1), jnp.float32)], axis=1)
    ea1 = jnp.concatenate([ea1, jnp.zeros((pad, NK), jnp.float32)], axis=0)

    def wcat(nnW, nnb, d_in):
        w = nnW.reshape(D_EDGE, d_in, H).transpose(1, 0, 2).reshape(d_in, -1)
        w = jnp.concatenate([w, nnb.reshape(d_in, H)], axis=1)
        return jnp.pad(w, ((0, 0), (0, KW - w.shape[1]))).astype(jnp.bfloat16)

    wall1 = wcat(nn1_W, nn1_b, D_NODE)
    wall2 = wcat(nn2_W, nn2_b, H)
    rmat = jnp.repeat(jnp.eye(NK, dtype=jnp.bfloat16), H, axis=1)
    zeros_np = jnp.zeros((ROWS_PER_TILE, H), jnp.float32)
    batch2d = batch.astype(jnp.int32).reshape(1, N)

    xj = _make_gather(D_NODE)(x, src_p)
    msg1 = _msg64(xj, ea1, wall1, rmat)
    agg1 = _make_scatter()(msg1, dst_p, zeros_np)
    h1 = _update1(x, agg1, root1, bias1.reshape(1, H))

    hj = _make_gather(H)(h1, src_p)
    msg2 = _msg32(hj, ea1, wall2, rmat)
    agg2 = _make_scatter()(msg2, dst_p, zeros_np)

    return _final(h1, agg2, root2, bias2.reshape(1, H),
                  batch2d, lin1_W, lin1_b.reshape(1, H // 2),
                  lin2_W, lin2_b.reshape(1, 1))
